# Initial kernel scaffold; baseline (speedup 1.0000x reference)
import jax
import jax.numpy as jnp
from jax.experimental import pallas as pl


def kernel(inp, gate, weight):
    raise NotImplementedError("write your pallas kernel here")


# speedup vs baseline: 2.5502x; 2.5502x over previous
import jax
import jax.numpy as jnp
from jax.experimental import pallas as pl

NUM_EXPERT = 64
IN_FEAT = 768
OUT_FEAT = 768
N_TOKENS = 128


def _moe_kernel(gate_ref, inp_ref, w_ref, out_ref):
    e = pl.program_id(0)

    @pl.when(e == 0)
    def _init():
        out_ref[...] = jnp.zeros_like(out_ref)

    mask = gate_ref[...] == e
    x = jnp.where(mask, inp_ref[...], 0.0)
    partial = jax.lax.dot_general(
        x, w_ref[0],
        (((1,), (1,)), ((), ())),
        preferred_element_type=jnp.float32,
    )
    out_ref[...] += partial


def kernel(inp, gate, weight):
    gate2d = gate.reshape(N_TOKENS, 1)
    return pl.pallas_call(
        _moe_kernel,
        grid=(NUM_EXPERT,),
        in_specs=[
            pl.BlockSpec((N_TOKENS, 1), lambda e: (0, 0)),
            pl.BlockSpec((N_TOKENS, IN_FEAT), lambda e: (0, 0)),
            pl.BlockSpec((1, OUT_FEAT, IN_FEAT), lambda e: (e, 0, 0)),
        ],
        out_specs=pl.BlockSpec((N_TOKENS, OUT_FEAT), lambda e: (0, 0)),
        out_shape=jax.ShapeDtypeStruct((N_TOKENS, OUT_FEAT), jnp.float32),
    )(gate2d, inp, weight)
